# baseline (speedup 1.0000x reference)
import functools

import jax
import jax.numpy as jnp
from jax import lax
from jax.experimental import pallas as pl
from jax.experimental.pallas import tpu as pltpu
from jax.experimental.pallas import tpu_sc as plsc

N = 10000
E = 320000
IN_DIM = 128
HID = 32
OUT = 16

CH = 128
NTILES = 32
CHUNKS = E // CH
BASE = CHUNKS // NTILES
NEXTRA = CHUNKS - BASE * NTILES
OUTN = 10240
ROWS_PER_SUBCORE = OUTN // 16
YSTRIPE = N // 16


def _make_agg(D):
  mesh = plsc.VectorSubcoreMesh(core_axis_name="c", subcore_axis_name="s")

  @functools.partial(
      pl.kernel,
      out_type=jax.ShapeDtypeStruct((2, OUTN, D), jnp.float32),
      mesh=mesh,
      scratch_types=[
          pltpu.VMEM((BASE, CH), jnp.int32),
          pltpu.VMEM((BASE, CH), jnp.int32),
          pltpu.VMEM((1, CH), jnp.int32),
          pltpu.VMEM((1, CH), jnp.int32),
          pltpu.VMEM((6, CH, D), jnp.float32),
          pltpu.VMEM_SHARED((OUTN, D), jnp.float32),
          pltpu.VMEM_SHARED((N, D), jnp.float32),
          [pltpu.SemaphoreType.DMA] * 6,
          [pltpu.SemaphoreType.DMA] * 6,
      ],
      compiler_params=pltpu.CompilerParams(use_tc_tiling_on_sc=False),
  )
  def agg(y_hbm, ei3_hbm, zero_hbm, out_hbm, rowi, coli, rowx, colx,
          gb, acc, y_s, gsem, ssem):
    c = lax.axis_index("c")
    s = lax.axis_index("s")
    tile = c * 16 + s

    @pl.when(s == 0)
    def _init():
      pltpu.sync_copy(zero_hbm, acc)

    pltpu.sync_copy(y_hbm.at[pl.ds(s * YSTRIPE, YSTRIPE)],
                    y_s.at[pl.ds(s * YSTRIPE, YSTRIPE)])

    start = tile * BASE
    pltpu.sync_copy(ei3_hbm.at[0, pl.ds(start, BASE)], rowi)
    pltpu.sync_copy(ei3_hbm.at[1, pl.ds(start, BASE)], coli)

    @pl.when(tile < NEXTRA)
    def _load_extra():
      pltpu.sync_copy(ei3_hbm.at[0, pl.ds(BASE * NTILES + tile, 1)], rowx)
      pltpu.sync_copy(ei3_hbm.at[1, pl.ds(BASE * NTILES + tile, 1)], colx)

    plsc.subcore_barrier()

    pltpu.async_copy(y_s.at[coli.at[0]], gb.at[0], gsem[0])
    pltpu.async_copy(y_s.at[coli.at[1]], gb.at[1], gsem[1])
    pltpu.async_copy(y_s.at[coli.at[2]], gb.at[2], gsem[2])

    def body(i6, carry):
      j0 = i6 * 6
      for k in range(6):
        j = j0 + k
        bn = (k + 3) % 6

        pltpu.make_async_copy(y_s.at[coli.at[j]], gb.at[k], gsem[k]).wait()
        pltpu.async_copy(gb.at[k], acc.at[rowi.at[j]], ssem[k], add=True)

        @pl.when(j - 3 >= 0)
        def _():
          pltpu.make_async_copy(gb.at[bn], acc.at[rowi.at[j - 3]],
                                ssem[bn]).wait()

        @pl.when(j + 3 < BASE)
        def _():
          pltpu.async_copy(y_s.at[coli.at[j + 3]], gb.at[bn], gsem[bn])
      return carry

    lax.fori_loop(0, BASE // 6, body, 0)

    for k in range(3, 6):
      pltpu.make_async_copy(gb.at[k], acc.at[rowi.at[BASE - 6 + k]],
                            ssem[k]).wait()

    @pl.when(tile < NEXTRA)
    def _do_extra():
      pltpu.async_copy(y_s.at[colx.at[0]], gb.at[0], gsem[0]).wait()
      pltpu.sync_copy(gb.at[0], acc.at[rowx.at[0]], add=True)

    plsc.subcore_barrier()
    pltpu.sync_copy(
        acc.at[pl.ds(s * ROWS_PER_SUBCORE, ROWS_PER_SUBCORE)],
        out_hbm.at[c, pl.ds(s * ROWS_PER_SUBCORE, ROWS_PER_SUBCORE)])

  return agg


_agg32 = _make_agg(HID)
_agg16 = _make_agg(OUT)


_RB1 = 5000
_RB2 = 512


def _mm1_body(x_ref, w_ref, o_ref):
  o_ref[...] = lax.dot_general(
      x_ref[...], w_ref[...], (((1,), (1,)), ((), ())),
      preferred_element_type=jnp.float32)


def _mid_body(p_ref, b_ref, w_ref, o_ref):
  h = jnp.maximum(p_ref[0] + p_ref[1] + b_ref[...], 0.0)
  o_ref[...] = lax.dot_general(
      h, w_ref[...], (((1,), (0,)), ((), ())),
      preferred_element_type=jnp.float32)


def _fin_body(q_ref, b_ref, o_ref):
  o_ref[...] = q_ref[0] + q_ref[1] + b_ref[...]


def kernel(x, edge_index, W1, b1, W2, b2):
  ei3 = edge_index.reshape(2, CHUNKS, CH)
  zeros32 = jnp.zeros((OUTN, HID), jnp.float32)
  zeros16 = jnp.zeros((OUTN, OUT), jnp.float32)
  w2big = jnp.kron(jnp.eye(4, dtype=jnp.float32), W2.T)
  b1t = jnp.tile(b1, 4).reshape(1, 128)
  b2t = jnp.tile(b2, 8).reshape(1, 128)

  y = pl.pallas_call(
      _mm1_body,
      grid=(N // _RB1,),
      in_specs=[pl.BlockSpec((_RB1, IN_DIM), lambda i: (i, 0)),
                pl.BlockSpec((HID, IN_DIM), lambda i: (0, 0))],
      out_specs=pl.BlockSpec((_RB1, HID), lambda i: (i, 0)),
      out_shape=jax.ShapeDtypeStruct((N, HID), jnp.float32),
  )(x, W1)

  p = _agg32(y, ei3, zeros32)

  p128 = p.reshape(2, OUTN * HID // 128, 128)
  z64 = pl.pallas_call(
      _mid_body,
      grid=(OUTN * HID // 128 // _RB2,),
      in_specs=[pl.BlockSpec((2, _RB2, 128), lambda i: (0, i, 0)),
                pl.BlockSpec((1, 128), lambda i: (0, 0)),
                pl.BlockSpec((128, 64), lambda i: (0, 0))],
      out_specs=pl.BlockSpec((_RB2, 64), lambda i: (i, 0)),
      out_shape=jax.ShapeDtypeStruct((OUTN * HID // 128, 64), jnp.float32),
  )(p128, b1t, w2big)

  z = z64.reshape(OUTN, OUT)
  q = _agg16(z, ei3, zeros16)

  q128 = q.reshape(2, OUTN * OUT // 128, 128)
  o128 = pl.pallas_call(
      _fin_body,
      grid=(1,),
      in_specs=[pl.BlockSpec((2, OUTN * OUT // 128, 128), lambda i: (0, 0, 0)),
                pl.BlockSpec((1, 128), lambda i: (0, 0))],
      out_specs=pl.BlockSpec((OUTN * OUT // 128, 128), lambda i: (i, 0)),
      out_shape=jax.ShapeDtypeStruct((OUTN * OUT // 128, 128), jnp.float32),
  )(q128, b2t)

  return o128[:N * OUT // 128].reshape(N, OUT)


# speedup vs baseline: 1.0174x; 1.0174x over previous
import functools

import jax
import jax.numpy as jnp
from jax import lax
from jax.experimental import pallas as pl
from jax.experimental.pallas import tpu as pltpu
from jax.experimental.pallas import tpu_sc as plsc

N = 10000
E = 320000
IN_DIM = 128
HID = 32
OUT = 16

CH = 128
NTILES = 32
CHUNKS = E // CH
BASE = CHUNKS // NTILES
NEXTRA = CHUNKS - BASE * NTILES
OUTN = 10240
ROWS_PER_SUBCORE = OUTN // 16
YSTRIPE = N // 16


def _make_agg(D):
  mesh = plsc.VectorSubcoreMesh(core_axis_name="c", subcore_axis_name="s")

  @functools.partial(
      pl.kernel,
      out_type=jax.ShapeDtypeStruct((2, OUTN, D), jnp.float32),
      mesh=mesh,
      scratch_types=[
          pltpu.VMEM((BASE, CH), jnp.int32),
          pltpu.VMEM((BASE, CH), jnp.int32),
          pltpu.VMEM((1, CH), jnp.int32),
          pltpu.VMEM((1, CH), jnp.int32),
          pltpu.VMEM((6, CH, D), jnp.float32),
          pltpu.VMEM_SHARED((OUTN, D), jnp.float32),
          pltpu.VMEM_SHARED((N, D), jnp.float32),
          [pltpu.SemaphoreType.DMA] * 6,
          [pltpu.SemaphoreType.DMA] * 6,
      ],
      compiler_params=pltpu.CompilerParams(use_tc_tiling_on_sc=False),
  )
  def agg(y_hbm, ei3_hbm, zero_hbm, out_hbm, rowi, coli, rowx, colx,
          gb, acc, y_s, gsem, ssem):
    c = lax.axis_index("c")
    s = lax.axis_index("s")
    tile = c * 16 + s

    @pl.when(s == 0)
    def _init():
      pltpu.sync_copy(zero_hbm, acc)

    pltpu.sync_copy(y_hbm.at[pl.ds(s * YSTRIPE, YSTRIPE)],
                    y_s.at[pl.ds(s * YSTRIPE, YSTRIPE)])

    start = tile * BASE
    pltpu.sync_copy(ei3_hbm.at[0, pl.ds(start, BASE)], rowi)
    pltpu.sync_copy(ei3_hbm.at[1, pl.ds(start, BASE)], coli)

    @pl.when(tile < NEXTRA)
    def _load_extra():
      pltpu.sync_copy(ei3_hbm.at[0, pl.ds(BASE * NTILES + tile, 1)], rowx)
      pltpu.sync_copy(ei3_hbm.at[1, pl.ds(BASE * NTILES + tile, 1)], colx)

    plsc.subcore_barrier()

    pltpu.async_copy(y_s.at[coli.at[0]], gb.at[0], gsem[0])
    pltpu.async_copy(y_s.at[coli.at[1]], gb.at[1], gsem[1])
    pltpu.async_copy(y_s.at[coli.at[2]], gb.at[2], gsem[2])

    def body(i6, carry):
      j0 = i6 * 6
      for k in range(6):
        j = j0 + k
        bn = (k + 3) % 6

        pltpu.make_async_copy(y_s.at[coli.at[j]], gb.at[k], gsem[k]).wait()
        pltpu.async_copy(gb.at[k], acc.at[rowi.at[j]], ssem[k], add=True)

        @pl.when(j - 3 >= 0)
        def _():
          pltpu.make_async_copy(gb.at[bn], acc.at[rowi.at[j - 3]],
                                ssem[bn]).wait()

        @pl.when(j + 3 < BASE)
        def _():
          pltpu.async_copy(y_s.at[coli.at[j + 3]], gb.at[bn], gsem[bn])
      return carry

    lax.fori_loop(0, BASE // 6, body, 0)

    for k in range(3, 6):
      pltpu.make_async_copy(gb.at[k], acc.at[rowi.at[BASE - 6 + k]],
                            ssem[k]).wait()

    @pl.when(tile < NEXTRA)
    def _do_extra():
      pltpu.async_copy(y_s.at[colx.at[0]], gb.at[0], gsem[0]).wait()
      pltpu.sync_copy(gb.at[0], acc.at[rowx.at[0]], add=True)

    plsc.subcore_barrier()
    pltpu.sync_copy(
        acc.at[pl.ds(s * ROWS_PER_SUBCORE, ROWS_PER_SUBCORE)],
        out_hbm.at[c, pl.ds(s * ROWS_PER_SUBCORE, ROWS_PER_SUBCORE)])

  return agg


_agg32 = _make_agg(HID)
_agg16 = _make_agg(OUT)


_RB1 = 5000
_RB2 = 2560


def _mm1_body(x_ref, w_ref, o_ref):
  o_ref[...] = lax.dot_general(
      x_ref[...], w_ref[...], (((1,), (1,)), ((), ())),
      preferred_element_type=jnp.float32)


def _mid_body(p_ref, b_ref, w_ref, o_ref):
  h = jnp.maximum(p_ref[0] + p_ref[1] + b_ref[...], 0.0)
  o_ref[...] = lax.dot_general(
      h, w_ref[...], (((1,), (0,)), ((), ())),
      preferred_element_type=jnp.float32)


def _fin_body(q_ref, b_ref, o_ref):
  o_ref[...] = q_ref[0] + q_ref[1] + b_ref[...]


def kernel(x, edge_index, W1, b1, W2, b2):
  ei3 = edge_index.reshape(2, CHUNKS, CH)
  zeros32 = jnp.zeros((OUTN, HID), jnp.float32)
  zeros16 = jnp.zeros((OUTN, OUT), jnp.float32)
  w2big = jnp.kron(jnp.eye(4, dtype=jnp.float32), W2.T)
  b1t = jnp.tile(b1, 4).reshape(1, 128)
  b2t = jnp.tile(b2, 8).reshape(1, 128)

  y = pl.pallas_call(
      _mm1_body,
      grid=(N // _RB1,),
      in_specs=[pl.BlockSpec((_RB1, IN_DIM), lambda i: (i, 0)),
                pl.BlockSpec((HID, IN_DIM), lambda i: (0, 0))],
      out_specs=pl.BlockSpec((_RB1, HID), lambda i: (i, 0)),
      out_shape=jax.ShapeDtypeStruct((N, HID), jnp.float32),
  )(x, W1)

  p = _agg32(y, ei3, zeros32)

  p128 = p.reshape(2, OUTN * HID // 128, 128)
  z64 = pl.pallas_call(
      _mid_body,
      grid=(OUTN * HID // 128 // _RB2,),
      in_specs=[pl.BlockSpec((2, _RB2, 128), lambda i: (0, i, 0)),
                pl.BlockSpec((1, 128), lambda i: (0, 0)),
                pl.BlockSpec((128, 64), lambda i: (0, 0))],
      out_specs=pl.BlockSpec((_RB2, 64), lambda i: (i, 0)),
      out_shape=jax.ShapeDtypeStruct((OUTN * HID // 128, 64), jnp.float32),
  )(p128, b1t, w2big)

  z = z64.reshape(OUTN, OUT)
  q = _agg16(z, ei3, zeros16)

  q128 = q.reshape(2, OUTN * OUT // 128, 128)
  o128 = pl.pallas_call(
      _fin_body,
      grid=(1,),
      in_specs=[pl.BlockSpec((2, OUTN * OUT // 128, 128), lambda i: (0, 0, 0)),
                pl.BlockSpec((1, 128), lambda i: (0, 0))],
      out_specs=pl.BlockSpec((OUTN * OUT // 128, 128), lambda i: (i, 0)),
      out_shape=jax.ShapeDtypeStruct((OUTN * OUT // 128, 128), jnp.float32),
  )(q128, b2t)

  return o128[:N * OUT // 128].reshape(N, OUT)
